# baseline (speedup 1.0000x reference)
import functools

import jax
import jax.numpy as jnp
from jax import lax
from jax.experimental import pallas as pl
from jax.experimental.pallas import tpu as pltpu
from jax.experimental.pallas import tpu_sc as plsc

BATCH = 16384
NROWS = 1000000
D = 32
LANES = 16
NC = 2
NS = 16
NW = NC * NS
BPW = BATCH // NW
NGROUP = BPW // LANES
NELEM = D * BPW
ECHUNK = 128
NECHUNK = NELEM // ECHUNK


@functools.partial(
    pl.kernel,
    out_type=jax.ShapeDtypeStruct((BATCH,), jnp.float32),
    mesh=plsc.VectorSubcoreMesh(core_axis_name="c", subcore_axis_name="s"),
    compiler_params=pltpu.CompilerParams(
        needs_layout_passes=False, use_tc_tiling_on_sc=False
    ),
    scratch_types=[
        pltpu.VMEM((BPW,), jnp.int32),
        pltpu.VMEM((BPW,), jnp.int32),
        pltpu.VMEM((NECHUNK, ECHUNK), jnp.int32),
        pltpu.VMEM((NECHUNK, ECHUNK), jnp.int32),
        pltpu.VMEM((NELEM,), jnp.float32),
        pltpu.VMEM((NELEM,), jnp.float32),
        pltpu.VMEM((BPW,), jnp.float32),
        pltpu.SemaphoreType.DMA,
    ],
)
def _neumf_sc(users_hbm, items_hbm, ut_hbm, it_hbm, out_hbm,
              idx_u, idx_i, eidx_u, eidx_i, big_u, big_i, out_v, sem):
    wid = lax.axis_index("s") * NC + lax.axis_index("c")
    base = wid * BPW

    pltpu.sync_copy(users_hbm.at[pl.ds(base, BPW)], idx_u)
    pltpu.sync_copy(items_hbm.at[pl.ds(base, BPW)], idx_i)

    def build(d, carry):
        off = d * NROWS
        def chunk(c, carry2):
            pos = pl.multiple_of(c * LANES, LANES)
            r_u = idx_u[pl.ds(pos, LANES)]
            r_i = idx_i[pl.ds(pos, LANES)]
            j = d * (BPW // ECHUNK) + c // (ECHUNK // LANES)
            kk = pl.multiple_of((c % (ECHUNK // LANES)) * LANES, LANES)
            eidx_u[j, pl.ds(kk, LANES)] = r_u + off
            eidx_i[j, pl.ds(kk, LANES)] = r_i + off
            return carry2
        lax.fori_loop(0, NGROUP, chunk, 0)
        return carry
    lax.fori_loop(0, D, build, 0)

    copies = []
    for j in range(NECHUNK):
        dst = pl.ds(j * ECHUNK, ECHUNK)
        copies.append(pltpu.async_copy(ut_hbm.at[eidx_u.at[j]], big_u.at[dst], sem))
        copies.append(pltpu.async_copy(it_hbm.at[eidx_i.at[j]], big_i.at[dst], sem))
    for cp in copies:
        cp.wait()

    def group(g, carry):
        col0 = pl.multiple_of(g * LANES, LANES)
        acc = jnp.zeros((LANES,), jnp.float32)
        for d in range(D):
            s = pl.multiple_of(d * BPW + g * LANES, LANES)
            acc = acc + big_u[pl.ds(s, LANES)] * big_i[pl.ds(s, LANES)]
        out_v[pl.ds(col0, LANES)] = acc
        return carry

    lax.fori_loop(0, NGROUP, group, 0)

    pltpu.sync_copy(out_v, out_hbm.at[pl.ds(base, BPW)])


def kernel(users, items, user_table, item_table):
    ut_flat = user_table.T.reshape(D * NROWS)
    it_flat = item_table.T.reshape(D * NROWS)
    return _neumf_sc(users.astype(jnp.int32), items.astype(jnp.int32),
                     ut_flat, it_flat)


# speedup vs baseline: 20.1560x; 20.1560x over previous
import functools

import jax
import jax.numpy as jnp
from jax import lax
from jax.experimental import pallas as pl
from jax.experimental.pallas import tpu as pltpu
from jax.experimental.pallas import tpu_sc as plsc

BATCH = 16384
NROWS = 1000000
D = 32
LANES = 16
NC = 2
NS = 16
NW = NC * NS
BPW = BATCH // NW
SB = 4
NSB = BPW // SB
SUPER = 16
NSUPER = BPW // SUPER


@functools.partial(
    pl.kernel,
    out_type=jax.ShapeDtypeStruct((BATCH,), jnp.float32),
    mesh=plsc.VectorSubcoreMesh(core_axis_name="c", subcore_axis_name="s"),
    compiler_params=pltpu.CompilerParams(
        needs_layout_passes=False, disable_bounds_checks=True
    ),
    scratch_types=[
        pltpu.VMEM((BPW + SUPER,), jnp.int32),
        pltpu.VMEM((BPW + SUPER,), jnp.int32),
        pltpu.VMEM((2, SB, 2, 4, 8, 128), jnp.float32),
        pltpu.VMEM((BPW,), jnp.float32),
        pltpu.SemaphoreType.DMA,
        pltpu.SemaphoreType.DMA,
    ],
)
def _neumf_sc(users_hbm, items_hbm, utv_hbm, itv_hbm, out_hbm,
              idx_u, idx_i, ring, out_v, sem_a, sem_b):
    wid = lax.axis_index("s") * NC + lax.axis_index("c")
    base = wid * BPW

    pltpu.sync_copy(users_hbm.at[pl.ds(base, BPW)], idx_u.at[pl.ds(0, BPW)])
    pltpu.sync_copy(items_hbm.at[pl.ds(base, BPW)], idx_i.at[pl.ds(0, BPW)])
    idx_u[pl.ds(BPW, SUPER)] = jnp.zeros((SUPER,), jnp.int32)
    idx_i[pl.ds(BPW, SUPER)] = jnp.zeros((SUPER,), jnp.int32)

    lanes = lax.iota(jnp.int32, LANES)
    db_lo = lanes // 8
    db_hi = db_lo + 2
    d8 = lanes % 8
    is_last = lanes == LANES - 1

    def issue(rv_u, rv_i, k0, slot, sem):
        for k in range(SB):
            ru = rv_u[k0 + k]
            ri = rv_i[k0 + k]
            tu = pl.multiple_of((ru // 128) * 128, 128)
            ti = pl.multiple_of((ri // 128) * 128, 128)
            for db in range(4):
                pltpu.async_copy(
                    utv_hbm.at[pl.ds(db * 8, 8), pl.ds(tu, 128)],
                    ring.at[slot, k, 0, db], sem)
                pltpu.async_copy(
                    itv_hbm.at[pl.ds(db * 8, 8), pl.ds(ti, 128)],
                    ring.at[slot, k, 1, db], sem)

    def drain(sem):
        for _ in range(SB * 8):
            pltpu.make_async_copy(
                utv_hbm.at[pl.ds(0, 8), pl.ds(0, 128)],
                ring.at[0, 0, 0, 0], sem).wait()

    def compute(rv_u, rv_i, k0, slot, nb, j):
        for k in range(SB):
            lu = jnp.full((LANES,), rv_u[k0 + k] % 128, jnp.int32)
            li = jnp.full((LANES,), rv_i[k0 + k] % 128, jnp.int32)
            slot_v = jnp.full((LANES,), slot, jnp.int32)
            k_v = jnp.full((LANES,), k, jnp.int32)
            t0 = jnp.zeros((LANES,), jnp.int32)
            t1 = jnp.full((LANES,), 1, jnp.int32)
            u_lo = plsc.load_gather(ring, [slot_v, k_v, t0, db_lo, d8, lu])
            u_hi = plsc.load_gather(ring, [slot_v, k_v, t0, db_hi, d8, lu])
            i_lo = plsc.load_gather(ring, [slot_v, k_v, t1, db_lo, d8, li])
            i_hi = plsc.load_gather(ring, [slot_v, k_v, t1, db_hi, d8, li])
            acc = u_lo * i_lo + u_hi * i_hi
            tot = jnp.cumsum(acc)
            pos = nb * SUPER + j * SB + k
            plsc.store_scatter(out_v, [jnp.full((LANES,), pos, jnp.int32)],
                               tot, mask=is_last)

    rv_u0 = idx_u[pl.ds(0, LANES)]
    rv_i0 = idx_i[pl.ds(0, LANES)]
    issue(rv_u0, rv_i0, 0, 0, sem_a)

    def super_batch(nb, carry):
        off = pl.multiple_of(nb * SUPER, SUPER)
        rv_u = idx_u[pl.ds(off, LANES)]
        rv_i = idx_i[pl.ds(off, LANES)]
        off_n = pl.multiple_of(nb * SUPER + SUPER, SUPER)
        rv_un = idx_u[pl.ds(off_n, LANES)]
        rv_in = idx_i[pl.ds(off_n, LANES)]
        for j in range(SUPER // SB):
            slot, sem = j % 2, (sem_a, sem_b)[j % 2]
            slot_n, sem_n = (j + 1) % 2, (sem_a, sem_b)[(j + 1) % 2]
            if j + 1 < SUPER // SB:
                issue(rv_u, rv_i, (j + 1) * SB, slot_n, sem_n)
            else:
                issue(rv_un, rv_in, 0, slot_n, sem_n)
            drain(sem)
            compute(rv_u, rv_i, j * SB, slot, nb, j)
        return carry

    lax.fori_loop(0, NSUPER, super_batch, 0)
    drain(sem_a)

    pltpu.sync_copy(out_v, out_hbm.at[pl.ds(base, BPW)])


def kernel(users, items, user_table, item_table):
    return _neumf_sc(users.astype(jnp.int32), items.astype(jnp.int32),
                     user_table.T, item_table.T)
